# baseline (speedup 1.0000x reference)
import functools

import jax
import jax.numpy as jnp
from jax import lax
from jax.experimental import pallas as pl
from jax.experimental.pallas import tpu as pltpu
from jax.experimental.pallas import tpu_sc as plsc

FILTER = -1e9
K = 64
TOP_P = 0.9
MIN_KEEP = 3.0
NC = 2
NS = 16
ROWS_PER_TILE = 2


def _sortv(x):
    return jnp.sort(x)


def _rev(x):
    return jnp.flip(x, 0)


def _merge16(a, b):
    rb = _rev(b)
    return _sortv(jnp.minimum(a, rb)), _sortv(jnp.maximum(a, rb))


def _clean32(u, v):
    return _sortv(jnp.minimum(u, v)), _sortv(jnp.maximum(u, v))


def _merge32(a0, a1, b0, b1):
    r0, r1 = _rev(b1), _rev(b0)
    lo0, lo1 = jnp.minimum(a0, r0), jnp.minimum(a1, r1)
    hi0, hi1 = jnp.maximum(a0, r0), jnp.maximum(a1, r1)
    l0, l1 = _clean32(lo0, lo1)
    h0, h1 = _clean32(hi0, hi1)
    return l0, l1, h0, h1


def _clean64(h0, h1, h2, h3):
    a0, a1 = jnp.minimum(h0, h2), jnp.minimum(h1, h3)
    b0, b1 = jnp.maximum(h0, h2), jnp.maximum(h1, h3)
    l0, l1 = _clean32(a0, a1)
    u0, u1 = _clean32(b0, b1)
    return l0, l1, u0, u1


def _sc_body(x_hbm, out_hbm, row_v, pend_v, out_v):
    wid = lax.axis_index("s") * NC + lax.axis_index("c")
    C = x_hbm.shape[1]
    ngroups = C // 256

    def one_row(row):
        pltpu.sync_copy(x_hbm.at[row], row_v)

        def reset_pend():
            sent = jnp.full((16,), FILTER, jnp.float32)
            for off in range(0, 192, 16):
                pend_v[off:off + 16] = sent

        def make_flush(nb):
            def fl(carry):
                t, pc, t0, t1, t2, t3 = carry
                for b in range(nb):
                    o = b * 64
                    p0 = _sortv(pend_v[o:o + 16])
                    p1 = _sortv(pend_v[o + 16:o + 32])
                    p2 = _sortv(pend_v[o + 32:o + 48])
                    p3 = _sortv(pend_v[o + 48:o + 64])
                    l0, h0 = _merge16(p0, p1)
                    l1, h1 = _merge16(p2, p3)
                    q0, q1, q2, q3 = _merge32(l0, h0, l1, h1)
                    r0, r1, r2, r3 = _rev(q3), _rev(q2), _rev(q1), _rev(q0)
                    t0, t1, t2, t3 = _clean64(
                        jnp.maximum(t0, r0), jnp.maximum(t1, r1),
                        jnp.maximum(t2, r2), jnp.maximum(t3, r3))
                reset_pend()
                return jnp.min(t0), jnp.int32(0), t0, t1, t2, t3
            return fl

        def tiered_flush(carry):
            def two_or_three(c):
                return lax.cond(c[1] <= 128, make_flush(2), make_flush(3), c)
            return lax.cond(carry[1] <= 64, make_flush(1), two_or_three, carry)

        def group(i, carry):
            t, pc, t0, t1, t2, t3 = carry
            base = i * 256
            xs = [row_v[pl.ds(base + 16 * j, 16)] for j in range(16)]
            mx = xs[0]
            for xx in xs[1:]:
                mx = jnp.maximum(mx, xx)
            anyc = plsc.all_reduce_population_count(mx > t)[0]

            def has_candidates(carry):
                t, pc, t0, t1, t2, t3 = carry
                for half in (xs[:8], xs[8:]):
                    for xx in half:
                        mask = xx > t
                        plsc.store_compressed(
                            pend_v.at[pl.ds(pc, 16)], xx, mask=mask)
                        pc = pc + plsc.all_reduce_population_count(mask)[0]
                    carry2 = (t, pc, t0, t1, t2, t3)
                    t, pc, t0, t1, t2, t3 = lax.cond(
                        pc >= 48, tiered_flush, lambda c: c, carry2)
                return t, pc, t0, t1, t2, t3

            return lax.cond(anyc > 0, has_candidates, lambda c: c, carry)

        sent = jnp.full((16,), FILTER, jnp.float32)
        reset_pend()
        init = (jnp.float32(FILTER), jnp.int32(0), sent, sent, sent, sent)
        carry = lax.fori_loop(0, ngroups, group, init)
        _, _, t0, t1, t2, t3 = make_flush(3)(carry)
        out_v[0:16] = _rev(t3)
        out_v[16:32] = _rev(t2)
        out_v[32:48] = _rev(t1)
        out_v[48:64] = _rev(t0)
        pltpu.sync_copy(out_v, out_hbm.at[row])

    for rr in range(ROWS_PER_TILE):
        one_row(wid * ROWS_PER_TILE + rr)


def _sc_topk(x):
    B, C = x.shape
    fn = functools.partial(
        pl.kernel,
        mesh=plsc.VectorSubcoreMesh(core_axis_name="c", subcore_axis_name="s"),
        compiler_params=pltpu.CompilerParams(needs_layout_passes=False),
        out_type=jax.ShapeDtypeStruct((B, K), jnp.float32),
        scratch_types=[
            pltpu.VMEM((C,), jnp.float32),
            pltpu.VMEM((192,), jnp.float32),
            pltpu.VMEM((K,), jnp.float32),
        ],
    )(_sc_body)
    return fn(x)


def _tc_body(x_ref, tv_ref, filt_ref, lp_ref):
    x = x_ref[...]
    tv = tv_ref[...]
    R = tv.shape[0]

    def psum(a):
        for sh in (1, 2, 4, 8, 16, 32):
            a = a + jnp.concatenate(
                [jnp.zeros((R, sh), a.dtype), a[:, : K - sh]], axis=1)
        return a

    v0 = tv[:, 0:1]
    w = jnp.exp(tv - v0)
    cumw = psum(w)
    s_all = cumw[:, K - 1:K]
    below = (cumw / s_all) <= TOP_P
    r = jnp.sum(below.astype(jnp.float32), axis=1, keepdims=True)
    m = jnp.maximum(r + 1.0, MIN_KEEP)
    pos = jax.lax.broadcasted_iota(jnp.int32, (1, K), 1).astype(jnp.float32)
    keepmask = pos < m
    thresh = jnp.min(jnp.where(keepmask, tv, jnp.inf), axis=1, keepdims=True)
    s_kept = jnp.sum(w * keepmask.astype(jnp.float32), axis=1, keepdims=True)
    lse = v0 + jnp.log(s_kept)
    filt = jnp.where(x >= thresh, x, FILTER)
    filt_ref[...] = filt
    lp_ref[...] = filt - lse


def kernel(logits, top_k):
    del top_k
    B, V = logits.shape
    C = (V + 2047) // 2048 * 2048
    x = jnp.pad(logits, ((0, 0), (0, C - V)), constant_values=FILTER)
    tv = _sc_topk(x)
    R = 8
    CB = C // 8
    filt, lp = pl.pallas_call(
        _tc_body,
        grid=(B // R, C // CB),
        in_specs=[
            pl.BlockSpec((R, CB), lambda i, j: (i, j)),
            pl.BlockSpec((R, K), lambda i, j: (i, 0)),
        ],
        out_specs=[
            pl.BlockSpec((R, CB), lambda i, j: (i, j)),
            pl.BlockSpec((R, CB), lambda i, j: (i, j)),
        ],
        out_shape=[
            jax.ShapeDtypeStruct((B, C), jnp.float32),
            jax.ShapeDtypeStruct((B, C), jnp.float32),
        ],
    )(x, tv)
    return filt[:, :V], lp[:, :V]


# speedup vs baseline: 1.0807x; 1.0807x over previous
import functools

import jax
import jax.numpy as jnp
from jax import lax
from jax.experimental import pallas as pl
from jax.experimental.pallas import tpu as pltpu
from jax.experimental.pallas import tpu_sc as plsc

FILTER = -1e9
K = 64
TOP_P = 0.9
MIN_KEEP = 3.0
NC = 2
NS = 16
ROWS_PER_TILE = 2


def _sortv(x):
    return jnp.sort(x)


def _rev(x):
    return jnp.flip(x, 0)


def _merge16(a, b):
    rb = _rev(b)
    return _sortv(jnp.minimum(a, rb)), _sortv(jnp.maximum(a, rb))


def _clean32(u, v):
    return _sortv(jnp.minimum(u, v)), _sortv(jnp.maximum(u, v))


def _merge32(a0, a1, b0, b1):
    r0, r1 = _rev(b1), _rev(b0)
    lo0, lo1 = jnp.minimum(a0, r0), jnp.minimum(a1, r1)
    hi0, hi1 = jnp.maximum(a0, r0), jnp.maximum(a1, r1)
    l0, l1 = _clean32(lo0, lo1)
    h0, h1 = _clean32(hi0, hi1)
    return l0, l1, h0, h1


def _clean64(h0, h1, h2, h3):
    a0, a1 = jnp.minimum(h0, h2), jnp.minimum(h1, h3)
    b0, b1 = jnp.maximum(h0, h2), jnp.maximum(h1, h3)
    l0, l1 = _clean32(a0, a1)
    u0, u1 = _clean32(b0, b1)
    return l0, l1, u0, u1


def _sc_body(x_hbm, out_hbm, row_v, pend_v, out_v):
    wid = lax.axis_index("s") * NC + lax.axis_index("c")
    C = x_hbm.shape[1]
    ngroups = C // 256

    def one_row(row):
        pltpu.sync_copy(x_hbm.at[row], row_v)

        def reset_pend():
            sent = jnp.full((16,), FILTER, jnp.float32)
            for off in range(0, 192, 16):
                pend_v[off:off + 16] = sent

        def make_flush(nb):
            def fl(carry):
                t, pc, t0, t1, t2, t3 = carry
                for b in range(nb):
                    o = b * 64
                    p0 = _sortv(pend_v[o:o + 16])
                    p1 = _sortv(pend_v[o + 16:o + 32])
                    p2 = _sortv(pend_v[o + 32:o + 48])
                    p3 = _sortv(pend_v[o + 48:o + 64])
                    l0, h0 = _merge16(p0, p1)
                    l1, h1 = _merge16(p2, p3)
                    q0, q1, q2, q3 = _merge32(l0, h0, l1, h1)
                    r0, r1, r2, r3 = _rev(q3), _rev(q2), _rev(q1), _rev(q0)
                    t0, t1, t2, t3 = _clean64(
                        jnp.maximum(t0, r0), jnp.maximum(t1, r1),
                        jnp.maximum(t2, r2), jnp.maximum(t3, r3))
                reset_pend()
                return jnp.min(t0), jnp.int32(0), t0, t1, t2, t3
            return fl

        def tiered_flush(carry):
            def two_or_three(c):
                return lax.cond(c[1] <= 128, make_flush(2), make_flush(3), c)
            return lax.cond(carry[1] <= 64, make_flush(1), two_or_three, carry)

        def group(i, carry):
            t, pc, t0, t1, t2, t3 = carry
            base = i * 256
            xs = [row_v[pl.ds(base + 16 * j, 16)] for j in range(16)]
            mx = xs[0]
            for xx in xs[1:]:
                mx = jnp.maximum(mx, xx)
            anyc = plsc.all_reduce_population_count(mx > t)[0]

            def has_candidates(carry):
                t, pc, t0, t1, t2, t3 = carry
                for half in (xs[:8], xs[8:]):
                    for xx in half:
                        mask = xx > t
                        plsc.store_compressed(
                            pend_v.at[pl.ds(pc, 16)], xx, mask=mask)
                        pc = pc + plsc.all_reduce_population_count(mask)[0]
                    carry2 = (t, pc, t0, t1, t2, t3)
                    t, pc, t0, t1, t2, t3 = lax.cond(
                        pc >= 48, tiered_flush, lambda c: c, carry2)
                return t, pc, t0, t1, t2, t3

            return lax.cond(anyc > 0, has_candidates, lambda c: c, carry)

        sent = jnp.full((16,), FILTER, jnp.float32)
        reset_pend()
        init = (jnp.float32(FILTER), jnp.int32(0), sent, sent, sent, sent)
        carry = lax.fori_loop(0, ngroups, group, init)
        _, _, t0, t1, t2, t3 = make_flush(3)(carry)
        out_v[0:16] = _rev(t3)
        out_v[16:32] = _rev(t2)
        out_v[32:48] = _rev(t1)
        out_v[48:64] = _rev(t0)
        pltpu.sync_copy(out_v, out_hbm.at[row])

    for rr in range(ROWS_PER_TILE):
        one_row(wid * ROWS_PER_TILE + rr)


def _sc_topk(x):
    B, C = x.shape
    fn = functools.partial(
        pl.kernel,
        mesh=plsc.VectorSubcoreMesh(core_axis_name="c", subcore_axis_name="s"),
        compiler_params=pltpu.CompilerParams(needs_layout_passes=False),
        out_type=jax.ShapeDtypeStruct((B, K), jnp.float32),
        scratch_types=[
            pltpu.VMEM((C,), jnp.float32),
            pltpu.VMEM((192,), jnp.float32),
            pltpu.VMEM((K,), jnp.float32),
        ],
    )(_sc_body)
    return fn(x)


def _tc_body(x_ref, tv_ref, filt_ref, lp_ref, th_ref, lse_ref):
    @pl.when(pl.program_id(1) == 0)
    def _nucleus():
        tv = tv_ref[...]
        R = tv.shape[0]

        def psum(a):
            for sh in (1, 2, 4, 8, 16, 32):
                a = a + jnp.concatenate(
                    [jnp.zeros((R, sh), a.dtype), a[:, : K - sh]], axis=1)
            return a

        v0 = tv[:, 0:1]
        w = jnp.exp(tv - v0)
        cumw = psum(w)
        s_all = cumw[:, K - 1:K]
        below = (cumw / s_all) <= TOP_P
        r = jnp.sum(below.astype(jnp.float32), axis=1, keepdims=True)
        m = jnp.maximum(r + 1.0, MIN_KEEP)
        pos = jax.lax.broadcasted_iota(jnp.int32, (1, K), 1)
        keepmask = pos.astype(jnp.float32) < m
        th_ref[...] = jnp.min(
            jnp.where(keepmask, tv, jnp.inf), axis=1, keepdims=True)
        s_kept = jnp.sum(w * keepmask.astype(jnp.float32), axis=1,
                         keepdims=True)
        lse_ref[...] = v0 + jnp.log(s_kept)

    x = x_ref[...]
    filt = jnp.where(x >= th_ref[...], x, FILTER)
    filt_ref[...] = filt
    lp_ref[...] = filt - lse_ref[...]


def kernel(logits, top_k):
    del top_k
    B, V = logits.shape
    C = (V + 2047) // 2048 * 2048
    x = jnp.pad(logits, ((0, 0), (0, C - V)), constant_values=FILTER)
    tv = _sc_topk(x)
    R = 8
    CB = C // 8
    filt, lp = pl.pallas_call(
        _tc_body,
        grid=(B // R, C // CB),
        in_specs=[
            pl.BlockSpec((R, CB), lambda i, j: (i, j)),
            pl.BlockSpec((R, K), lambda i, j: (i, 0)),
        ],
        out_specs=[
            pl.BlockSpec((R, CB), lambda i, j: (i, j)),
            pl.BlockSpec((R, CB), lambda i, j: (i, j)),
        ],
        out_shape=[
            jax.ShapeDtypeStruct((B, C), jnp.float32),
            jax.ShapeDtypeStruct((B, C), jnp.float32),
        ],
        scratch_shapes=[
            pltpu.VMEM((R, 1), jnp.float32),
            pltpu.VMEM((R, 1), jnp.float32),
        ],
    )(x, tv)
    return filt[:, :V], lp[:, :V]


# speedup vs baseline: 1.6864x; 1.5605x over previous
import functools

import jax
import jax.numpy as jnp
from jax import lax
from jax.experimental import pallas as pl
from jax.experimental.pallas import tpu as pltpu
from jax.experimental.pallas import tpu_sc as plsc

FILTER = -1e9
K = 64
TOP_P = 0.9
MIN_KEEP = 3.0
NC = 2
NS = 16
ROWS_PER_TILE = 2


def _sortv(x):
    return jnp.sort(x)


def _rev(x):
    return jnp.flip(x, 0)


def _merge16(a, b):
    rb = _rev(b)
    return _sortv(jnp.minimum(a, rb)), _sortv(jnp.maximum(a, rb))


def _clean32(u, v):
    return _sortv(jnp.minimum(u, v)), _sortv(jnp.maximum(u, v))


def _merge32(a0, a1, b0, b1):
    r0, r1 = _rev(b1), _rev(b0)
    lo0, lo1 = jnp.minimum(a0, r0), jnp.minimum(a1, r1)
    hi0, hi1 = jnp.maximum(a0, r0), jnp.maximum(a1, r1)
    l0, l1 = _clean32(lo0, lo1)
    h0, h1 = _clean32(hi0, hi1)
    return l0, l1, h0, h1


def _clean64(h0, h1, h2, h3):
    a0, a1 = jnp.minimum(h0, h2), jnp.minimum(h1, h3)
    b0, b1 = jnp.maximum(h0, h2), jnp.maximum(h1, h3)
    l0, l1 = _clean32(a0, a1)
    u0, u1 = _clean32(b0, b1)
    return l0, l1, u0, u1


def _sc_body(x_hbm, out_hbm, row_v, pend_v, out_v):
    wid = lax.axis_index("s") * NC + lax.axis_index("c")
    C = x_hbm.shape[1]
    ngroups = C // 256

    def one_row(row):
        pltpu.sync_copy(x_hbm.at[row], row_v)

        def reset_pend():
            sent = jnp.full((16,), FILTER, jnp.float32)
            for off in range(0, 192, 16):
                pend_v[off:off + 16] = sent

        def make_flush(nb):
            def fl(carry):
                t, pc, t0, t1, t2, t3 = carry
                for b in range(nb):
                    o = b * 64
                    p0 = _sortv(pend_v[o:o + 16])
                    p1 = _sortv(pend_v[o + 16:o + 32])
                    p2 = _sortv(pend_v[o + 32:o + 48])
                    p3 = _sortv(pend_v[o + 48:o + 64])
                    l0, h0 = _merge16(p0, p1)
                    l1, h1 = _merge16(p2, p3)
                    q0, q1, q2, q3 = _merge32(l0, h0, l1, h1)
                    r0, r1, r2, r3 = _rev(q3), _rev(q2), _rev(q1), _rev(q0)
                    t0, t1, t2, t3 = _clean64(
                        jnp.maximum(t0, r0), jnp.maximum(t1, r1),
                        jnp.maximum(t2, r2), jnp.maximum(t3, r3))
                reset_pend()
                return jnp.min(t0), jnp.int32(0), t0, t1, t2, t3
            return fl

        def tiered_flush(carry):
            def two_or_three(c):
                return lax.cond(c[1] <= 128, make_flush(2), make_flush(3), c)
            return lax.cond(carry[1] <= 64, make_flush(1), two_or_three, carry)

        def group(i, carry):
            t, pc, t0, t1, t2, t3 = carry
            base = i * 256
            xs = [row_v[pl.ds(base + 16 * j, 16)] for j in range(16)]
            mx = xs[0]
            for xx in xs[1:]:
                mx = jnp.maximum(mx, xx)
            anyc = plsc.all_reduce_population_count(mx > t)[0]

            def has_candidates(carry):
                t, pc, t0, t1, t2, t3 = carry
                for half in (xs[:8], xs[8:]):
                    for xx in half:
                        mask = xx > t
                        plsc.store_compressed(
                            pend_v.at[pl.ds(pc, 16)], xx, mask=mask)
                        pc = pc + plsc.all_reduce_population_count(mask)[0]
                    carry2 = (t, pc, t0, t1, t2, t3)
                    t, pc, t0, t1, t2, t3 = lax.cond(
                        pc >= 48, tiered_flush, lambda c: c, carry2)
                return t, pc, t0, t1, t2, t3

            return lax.cond(anyc > 0, has_candidates, lambda c: c, carry)

        sent = jnp.full((16,), FILTER, jnp.float32)
        reset_pend()
        init = (jnp.float32(FILTER), jnp.int32(0), sent, sent, sent, sent)
        carry = lax.fori_loop(0, ngroups, group, init)
        _, _, t0, t1, t2, t3 = make_flush(3)(carry)
        out_v[0:16] = _rev(t3)
        out_v[16:32] = _rev(t2)
        out_v[32:48] = _rev(t1)
        out_v[48:64] = _rev(t0)
        pltpu.sync_copy(out_v, out_hbm.at[row])

    for rr in range(ROWS_PER_TILE):
        one_row(wid * ROWS_PER_TILE + rr)


def _sc_topk(x):
    B, C = x.shape
    fn = functools.partial(
        pl.kernel,
        mesh=plsc.VectorSubcoreMesh(core_axis_name="c", subcore_axis_name="s"),
        compiler_params=pltpu.CompilerParams(needs_layout_passes=False),
        out_type=jax.ShapeDtypeStruct((B, K), jnp.float32),
        scratch_types=[
            pltpu.VMEM((C,), jnp.float32),
            pltpu.VMEM((192,), jnp.float32),
            pltpu.VMEM((K,), jnp.float32),
        ],
    )(_sc_body)
    return fn(x)


def _tc_body(x_ref, tv_ref, filt_ref, lp_ref):
    x = x_ref[...]
    tv = tv_ref[...]
    R = tv.shape[0]

    def psum(a):
        for sh in (1, 2, 4, 8, 16, 32):
            a = a + jnp.concatenate(
                [jnp.zeros((R, sh), a.dtype), a[:, : K - sh]], axis=1)
        return a

    v0 = tv[:, 0:1]
    w = jnp.exp(tv - v0)
    cumw = psum(w)
    s_all = cumw[:, K - 1:K]
    below = (cumw / s_all) <= TOP_P
    r = jnp.sum(below.astype(jnp.float32), axis=1, keepdims=True)
    m = jnp.maximum(r + 1.0, MIN_KEEP)
    pos = jax.lax.broadcasted_iota(jnp.int32, (1, K), 1).astype(jnp.float32)
    keepmask = pos < m
    thresh = jnp.min(jnp.where(keepmask, tv, jnp.inf), axis=1, keepdims=True)
    s_kept = jnp.sum(w * keepmask.astype(jnp.float32), axis=1, keepdims=True)
    lse = v0 + jnp.log(s_kept)
    filt = jnp.where(x >= thresh, x, FILTER)
    filt_ref[...] = filt
    lp_ref[...] = filt - lse


def kernel(logits, top_k):
    del top_k
    B, V = logits.shape
    C = (V + 255) // 256 * 256
    x = jnp.pad(logits, ((0, 0), (0, C - V)), constant_values=FILTER)
    tv = _sc_topk(x)
    R = 8
    filt, lp = pl.pallas_call(
        _tc_body,
        grid=(B // R,),
        in_specs=[
            pl.BlockSpec((R, C), lambda i: (i, 0)),
            pl.BlockSpec((R, K), lambda i: (i, 0)),
        ],
        out_specs=[
            pl.BlockSpec((R, C), lambda i: (i, 0)),
            pl.BlockSpec((R, C), lambda i: (i, 0)),
        ],
        out_shape=[
            jax.ShapeDtypeStruct((B, C), jnp.float32),
            jax.ShapeDtypeStruct((B, C), jnp.float32),
        ],
    )(x, tv)
    return filt[:, :V], lp[:, :V]
